# baseline (speedup 1.0000x reference)
import functools
import math

import jax
import jax.numpy as jnp
from jax import lax
from jax.experimental import pallas as pl
from jax.experimental.pallas import tpu as pltpu
from jax.experimental.pallas import tpu_sc as plsc

_R = 128
_K = 2
_CHUNK = _R * _K
_NBUF = 5
_P = 3


@functools.cache
def _make_gather(n_rows: int, d: int):
    info = plsc.get_sparse_core_info()
    nc, ns, nl = info.num_cores, info.num_subcores, info.num_lanes
    nw = nc * ns
    assert d % nl == 0
    per_w = n_rows // nw
    assert per_w * nw == n_rows
    groups = per_w // _CHUNK
    per_w_ir = per_w // _R
    assert groups * _CHUNK == per_w
    assert groups % _NBUF == 0 and groups >= 2 * _NBUF
    mesh = plsc.VectorSubcoreMesh(core_axis_name="c", subcore_axis_name="s")

    @functools.partial(
        pl.kernel,
        mesh=mesh,
        out_type=jax.ShapeDtypeStruct((n_rows, d), jnp.float32),
        scratch_types=[
            pltpu.VMEM((_NBUF, _K, _R), jnp.int32),
            pltpu.VMEM((_NBUF, _CHUNK, d), jnp.float32),
        ]
        + [pltpu.SemaphoreType.DMA] * (3 * _NBUF),
        compiler_params=pltpu.CompilerParams(use_tc_tiling_on_sc=False),
    )
    def gather_kernel(table_hbm, idx_hbm, out_hbm, idx_v, rows_v, *sems):
        gsem = sems[:_NBUF]
        osem = sems[_NBUF : 2 * _NBUF]
        isem = sems[2 * _NBUF :]
        wid = lax.axis_index("s") * nc + lax.axis_index("c")
        w_ir = wid * per_w_ir
        w_row = wid * per_w

        def fire_idx(c, s):
            pltpu.async_copy(
                idx_hbm.at[pl.ds(w_ir + c * _K, _K)], idx_v.at[s], isem[s])

        def wait_idx(s):
            pltpu.make_async_copy(
                idx_hbm.at[pl.ds(0, _K)], idx_v.at[s], isem[s]).wait()

        def fire_gather(c, s):
            for j in range(_K):
                pltpu.async_copy(
                    table_hbm.at[idx_v.at[s, j]],
                    rows_v.at[s, pl.ds(j * _R, _R)],
                    gsem[s])

        def wait_gather(s):
            pltpu.make_async_copy(
                table_hbm.at[pl.ds(0, _CHUNK)], rows_v.at[s], gsem[s]).wait()

        def fire_out(c, s):
            pltpu.async_copy(
                rows_v.at[s], out_hbm.at[pl.ds(w_row + c * _CHUNK, _CHUNK)],
                osem[s])

        def wait_out(s):
            pltpu.make_async_copy(
                out_hbm.at[pl.ds(0, _CHUNK)], rows_v.at[s], osem[s]).wait()

        def step(t, s, do_outwait, do_prefetch, do_idxfire):
            wait_gather(s)
            fire_out(t, s)
            if do_prefetch:
                ps = (s + _P) % _NBUF
                wait_idx(ps)
                if do_outwait:
                    wait_out(ps)
                fire_gather(t + _P, ps)
                if do_idxfire:
                    fire_idx(t + _P + 1, (s + _P + 1) % _NBUF)

        for c in range(_P):
            pltpu.sync_copy(idx_hbm.at[pl.ds(w_ir + c * _K, _K)], idx_v.at[c])
            fire_gather(c, c)
        fire_idx(_P, _P)

        for s in range(_NBUF):
            step(s, s, s >= _NBUF - _P, s + _P < groups, s + _P + 1 < groups)

        def block(b, carry):
            for s in range(_NBUF):
                step(b * _NBUF + s, s, True, True, True)
            return carry

        lax.fori_loop(1, groups // _NBUF - 1, block, 0)

        for s in range(_NBUF):
            t = groups - _NBUF + s
            step(t, s, True, t + _P < groups, t + _P + 1 < groups)

        for s in range(_NBUF):
            wait_out(s)

    return gather_kernel


def kernel(tokens, table):
    b, h = tokens.shape
    vocab, d = table.shape
    n_rows = b * h
    scaled = table * math.sqrt(d)
    idx2d = tokens.reshape(n_rows // _R, _R)
    out = _make_gather(n_rows, d)(scaled, idx2d)
    return out.reshape(b, h, d)


# speedup vs baseline: 1.0685x; 1.0685x over previous
import functools
import math

import jax
import jax.numpy as jnp
from jax import lax
from jax.experimental import pallas as pl
from jax.experimental.pallas import tpu as pltpu
from jax.experimental.pallas import tpu_sc as plsc

_R = 128
_K = 2
_CHUNK = _R * _K
_NBUF = 5
_P = 3


@functools.cache
def _make_gather(n_rows: int, d: int):
    info = plsc.get_sparse_core_info()
    nc, ns, nl = info.num_cores, info.num_subcores, info.num_lanes
    nw = nc * ns
    assert d % nl == 0
    per_w = n_rows // nw
    assert per_w * nw == n_rows
    groups = per_w // _CHUNK
    per_w_ir = per_w // _R
    assert groups * _CHUNK == per_w
    assert groups % _NBUF == 0 and groups >= 2 * _NBUF
    mesh = plsc.VectorSubcoreMesh(core_axis_name="c", subcore_axis_name="s")

    @functools.partial(
        pl.kernel,
        mesh=mesh,
        out_type=jax.ShapeDtypeStruct((n_rows, d), jnp.float32),
        scratch_types=[
            pltpu.VMEM((_NBUF, _K, _R), jnp.int32),
            pltpu.VMEM((_NBUF, _CHUNK, d), jnp.float32),
        ]
        + [pltpu.SemaphoreType.DMA] * (3 * _NBUF),
        compiler_params=pltpu.CompilerParams(use_tc_tiling_on_sc=False),
    )
    def gather_kernel(table_hbm, idx_hbm, out_hbm, idx_v, rows_v, *sems):
        gsem = sems[:_NBUF]
        osem = sems[_NBUF : 2 * _NBUF]
        isem = sems[2 * _NBUF :]
        wid = lax.axis_index("s") * nc + lax.axis_index("c")
        w_ir = wid * per_w_ir
        w_row = wid * per_w

        def fire_idx(c, s):
            pltpu.async_copy(
                idx_hbm.at[pl.ds(w_ir + c * _K, _K)], idx_v.at[s], isem[s])

        def wait_idx(s):
            pltpu.make_async_copy(
                idx_hbm.at[pl.ds(0, _K)], idx_v.at[s], isem[s]).wait()

        def fire_gather(c, s):
            for j in range(_K):
                pltpu.async_copy(
                    table_hbm.at[idx_v.at[s, j]],
                    rows_v.at[s, pl.ds(j * _R, _R)],
                    gsem[s])

        def wait_gather(s):
            pltpu.make_async_copy(
                table_hbm.at[pl.ds(0, _CHUNK)], rows_v.at[s], gsem[s]).wait()

        def fire_out(c, s):
            pltpu.async_copy(
                rows_v.at[s], out_hbm.at[pl.ds(w_row + c * _CHUNK, _CHUNK)],
                osem[s])

        def wait_out(s):
            pltpu.make_async_copy(
                out_hbm.at[pl.ds(0, _CHUNK)], rows_v.at[s], osem[s]).wait()

        def step(t, s, do_outwait, do_prefetch, do_idxfire):
            wait_gather(s)
            fire_out(t, s)
            if do_prefetch:
                ps = (s + _P) % _NBUF
                wait_idx(ps)
                if do_outwait:
                    wait_out(ps)
                fire_gather(t + _P, ps)
                if do_idxfire:
                    fire_idx(t + _P + 1, (s + _P + 1) % _NBUF)

        for c in range(_P):
            pltpu.sync_copy(idx_hbm.at[pl.ds(w_ir + c * _K, _K)], idx_v.at[c])
            fire_gather(c, c)
        fire_idx(_P, _P)

        for s in range(_NBUF):
            step(s, s, s >= _NBUF - _P, s + _P < groups, s + _P + 1 < groups)

        def block(b, carry):
            for s in range(_NBUF):
                step(b * _NBUF + s, s, True, True, True)
            return carry

        lax.fori_loop(1, groups // _NBUF - 1, block, 0)

        for s in range(_NBUF):
            t = groups - _NBUF + s
            step(t, s, True, t + _P < groups, t + _P + 1 < groups)

        for s in range(_NBUF):
            wait_out(s)

    return gather_kernel


def kernel(tokens, table):
    b, h = tokens.shape
    vocab, d = table.shape
    n_rows = b * h
    padded = jnp.pad(table * math.sqrt(d), ((0, 0), (0, d)))
    src = padded.reshape(2 * vocab, d)
    idx2d = (tokens * 2).reshape(n_rows // _R, _R)
    out = _make_gather(n_rows, d)(src, idx2d)
    return out.reshape(b, h, d)
